# Initial kernel scaffold; baseline (speedup 1.0000x reference)
import jax
import jax.numpy as jnp
from jax.experimental import pallas as pl


def kernel(x, edge_index, edge_attr, W1, b1, W2, b2, W3, b3, Wlin, blin):
    raise NotImplementedError("write your pallas kernel here")


# speedup vs baseline: 20.6439x; 20.6439x over previous
import functools

import jax
import jax.numpy as jnp
from jax import lax
from jax.experimental import pallas as pl
from jax.experimental.pallas import tpu as pltpu
from jax.experimental.pallas import tpu_sc as plsc

NC = 2
NS = 16
LANES = 16
NW = NC * NS
BATCH = 128

f32 = jnp.float32
i32 = jnp.int32


def _sc_mesh():
    return plsc.VectorSubcoreMesh(core_axis_name="c", subcore_axis_name="s",
                                  num_cores=NC, num_subcores=NS)


def _make_deg_kernel(NP, nb):
    NPS = NP // NS

    @functools.partial(
        pl.kernel,
        out_type=jax.ShapeDtypeStruct((NC, NP), f32),
        mesh=_sc_mesh(),
        compiler_params=pltpu.CompilerParams(needs_layout_passes=False,
                                             use_tc_tiling_on_sc=False),
        scratch_types=[
            pltpu.VMEM((nb, BATCH), i32),
            pltpu.VMEM((nb * BATCH,), f32),
            pltpu.VMEM_SHARED((NP,), f32),
        ],
    )
    def deg_kernel(dst_hbm, ew_hbm, z_hbm, out_hbm, dst_v, ew_v, acc):
        c = lax.axis_index("c")
        s = lax.axis_index("s")
        w = c * NS + s
        pltpu.sync_copy(z_hbm.at[pl.ds(s * NPS, NPS)], acc.at[pl.ds(s * NPS, NPS)])
        pltpu.sync_copy(dst_hbm.at[w], dst_v)
        pltpu.sync_copy(ew_hbm.at[w], ew_v)
        plsc.subcore_barrier()

        def body(g, carry):
            pltpu.sync_copy(ew_v.at[pl.ds(g * BATCH, BATCH)],
                            acc.at[dst_v.at[g]], add=True)
            return carry

        lax.fori_loop(0, nb, body, 0)
        plsc.subcore_barrier()
        pltpu.sync_copy(acc.at[pl.ds(s * NPS, NPS)],
                        out_hbm.at[c, pl.ds(s * NPS, NPS)])

    return deg_kernel


def _make_agg_kernel(NP, nb, H):
    NPS = NP // NS

    @functools.partial(
        pl.kernel,
        out_type=jax.ShapeDtypeStruct((NC, NP, H), f32),
        mesh=_sc_mesh(),
        compiler_params=pltpu.CompilerParams(needs_layout_passes=False,
                                             use_tc_tiling_on_sc=False),
        scratch_types=[
            pltpu.VMEM((nb, BATCH), i32),
            pltpu.VMEM((nb, BATCH), i32),
            pltpu.VMEM((nb * BATCH,), f32),
            pltpu.VMEM((BATCH, H), f32),
            pltpu.VMEM_SHARED((NP, H), f32),
            pltpu.SemaphoreType.DMA,
        ],
    )
    def agg_kernel(at_hbm, src_hbm, dst_hbm, ew_hbm, z_hbm, out_hbm,
                   src_v, dst_v, ew_v, rows_v, acc, sem):
        c = lax.axis_index("c")
        s = lax.axis_index("s")
        w = c * NS + s
        pltpu.sync_copy(z_hbm.at[pl.ds(s * NPS, NPS)], acc.at[pl.ds(s * NPS, NPS)])
        pltpu.sync_copy(src_hbm.at[w], src_v)
        pltpu.sync_copy(dst_hbm.at[w], dst_v)
        pltpu.sync_copy(ew_hbm.at[w], ew_v)
        plsc.subcore_barrier()

        def body(g, carry):
            pltpu.async_copy(at_hbm.at[src_v.at[g]], rows_v, sem).wait()
            base = g * BATCH
            for j in range(BATCH):
                scale = plsc.load_gather(
                    ew_v, [jnp.full((LANES,), base + j, i32)])
                rows_v[j, :] = rows_v[j, :] * scale
            pltpu.sync_copy(rows_v, acc.at[dst_v.at[g]], add=True)
            return carry

        lax.fori_loop(0, nb, body, 0)
        plsc.subcore_barrier()
        pltpu.sync_copy(acc.at[pl.ds(s * NPS, NPS)],
                        out_hbm.at[c, pl.ds(s * NPS, NPS)])

    return agg_kernel


def _tc_first(x, W1, d0, d1, R):
    N, D = x.shape
    H = W1.shape[1]

    def body(x_ref, w_ref, d0_ref, d1_ref, at_ref, dinv_ref):
        deg = d0_ref[...] + d1_ref[...] + 1.0
        dinv = lax.rsqrt(deg)
        a = jnp.dot(x_ref[...], w_ref[...], preferred_element_type=f32)
        at_ref[...] = a * dinv
        dinv_ref[...] = dinv

    return pl.pallas_call(
        body,
        grid=(N // R,),
        in_specs=[
            pl.BlockSpec((R, D), lambda i: (i, 0)),
            pl.BlockSpec((D, H), lambda i: (0, 0)),
            pl.BlockSpec((R, 1), lambda i: (i, 0)),
            pl.BlockSpec((R, 1), lambda i: (i, 0)),
        ],
        out_specs=[
            pl.BlockSpec((R, H), lambda i: (i, 0)),
            pl.BlockSpec((R, 1), lambda i: (i, 0)),
        ],
        out_shape=[jax.ShapeDtypeStruct((N, H), f32),
                   jax.ShapeDtypeStruct((N, 1), f32)],
    )(x, W1, d0, d1)


def _tc_combine(s0, s1, at, dinv, b, Wn, R):
    N, H = at.shape

    def body(s0_ref, s1_ref, at_ref, dinv_ref, b_ref, w_ref, out_ref):
        o = dinv_ref[...] * (s0_ref[...] + s1_ref[...] + at_ref[...]) + b_ref[...]
        h = jnp.dot(o, w_ref[...], preferred_element_type=f32)
        out_ref[...] = dinv_ref[...] * h

    return pl.pallas_call(
        body,
        grid=(N // R,),
        in_specs=[
            pl.BlockSpec((R, H), lambda i: (i, 0)),
            pl.BlockSpec((R, H), lambda i: (i, 0)),
            pl.BlockSpec((R, H), lambda i: (i, 0)),
            pl.BlockSpec((R, 1), lambda i: (i, 0)),
            pl.BlockSpec((1, H), lambda i: (0, 0)),
            pl.BlockSpec((H, H), lambda i: (0, 0)),
        ],
        out_specs=pl.BlockSpec((R, H), lambda i: (i, 0)),
        out_shape=jax.ShapeDtypeStruct((N, H), f32),
    )(s0, s1, at, dinv, b, Wn)


def _tc_final(s0, s1, at, dinv, b, Wlin, blin, R):
    N, H = at.shape
    grid = N // R

    def body(s0_ref, s1_ref, at_ref, dinv_ref, b_ref, wl_ref, bl_ref,
             out_ref, acc):
        i = pl.program_id(0)

        @pl.when(i == 0)
        def _():
            acc[...] = jnp.zeros_like(acc)

        o = dinv_ref[...] * (s0_ref[...] + s1_ref[...] + at_ref[...]) + b_ref[...]
        acc[...] += jnp.sum(o, axis=0, keepdims=True)

        @pl.when(i == grid - 1)
        def _():
            g = acc[...] * (1.0 / N)
            out_ref[...] = jnp.dot(g, wl_ref[...],
                                   preferred_element_type=f32) + bl_ref[...]

    return pl.pallas_call(
        body,
        grid=(grid,),
        in_specs=[
            pl.BlockSpec((R, H), lambda i: (i, 0)),
            pl.BlockSpec((R, H), lambda i: (i, 0)),
            pl.BlockSpec((R, H), lambda i: (i, 0)),
            pl.BlockSpec((R, 1), lambda i: (i, 0)),
            pl.BlockSpec((1, H), lambda i: (0, 0)),
            pl.BlockSpec((H, 1), lambda i: (0, 0)),
            pl.BlockSpec((1, 1), lambda i: (0, 0)),
        ],
        out_specs=pl.BlockSpec((1, 1), lambda i: (0, 0)),
        out_shape=jax.ShapeDtypeStruct((1, 1), f32),
        scratch_shapes=[pltpu.VMEM((1, H), f32)],
    )(s0, s1, at, dinv, b, Wlin, blin)


def kernel(x, edge_index, edge_attr, W1, b1, W2, b2, W3, b3, Wlin, blin):
    N, D = x.shape
    H = W1.shape[1]
    E = edge_index.shape[1]
    assert H == LANES

    NP = -(-N // (NS * LANES)) * (NS * LANES)
    EP = -(-E // (NW * BATCH)) * (NW * BATCH)
    nb = EP // (NW * BATCH)
    R = 2000
    assert N % R == 0

    src = edge_index[0]
    dst = edge_index[1]
    ew = edge_attr.astype(f32)
    pad = EP - E
    if pad:
        zi = jnp.zeros((pad,), i32)
        src = jnp.concatenate([src, zi])
        dst = jnp.concatenate([dst, zi])
        ew = jnp.concatenate([ew, jnp.zeros((pad,), f32)])
    src3 = src.reshape(NW, nb, BATCH)
    dst3 = dst.reshape(NW, nb, BATCH)
    ew3 = ew.reshape(NW, nb * BATCH)
    z1 = jnp.zeros((NP,), f32)
    z2 = jnp.zeros((NP, H), f32)

    deg_k = _make_deg_kernel(NP, nb)
    agg_k = _make_agg_kernel(NP, nb, H)

    degp = deg_k(dst3, ew3, z1)
    d0 = degp[0, :N, None]
    d1 = degp[1, :N, None]

    at1, dinv = _tc_first(x, W1, d0, d1, R)
    S = agg_k(at1, src3, dst3, ew3, z2)
    at2 = _tc_combine(S[0, :N], S[1, :N], at1, dinv, b1.reshape(1, H), W2, R)
    S = agg_k(at2, src3, dst3, ew3, z2)
    at3 = _tc_combine(S[0, :N], S[1, :N], at2, dinv, b2.reshape(1, H), W3, R)
    S = agg_k(at3, src3, dst3, ew3, z2)
    return _tc_final(S[0, :N], S[1, :N], at3, dinv, b3.reshape(1, H),
                     Wlin, blin.reshape(1, 1), R)


# speedup vs baseline: 21.7228x; 1.0523x over previous
import functools

import jax
import jax.numpy as jnp
from jax import lax
from jax.experimental import pallas as pl
from jax.experimental.pallas import tpu as pltpu
from jax.experimental.pallas import tpu_sc as plsc

NC = 2
NS = 16
LANES = 16
NW = NC * NS
BATCH = 128
NBUF = 4

f32 = jnp.float32
i32 = jnp.int32


def _sc_mesh():
    return plsc.VectorSubcoreMesh(core_axis_name="c", subcore_axis_name="s",
                                  num_cores=NC, num_subcores=NS)


def _make_deg_kernel(NP, nb):
    NPS = NP // NS

    @functools.partial(
        pl.kernel,
        out_type=jax.ShapeDtypeStruct((NC, NP), f32),
        mesh=_sc_mesh(),
        compiler_params=pltpu.CompilerParams(needs_layout_passes=False,
                                             use_tc_tiling_on_sc=False),
        scratch_types=[
            pltpu.VMEM((nb, BATCH), i32),
            pltpu.VMEM((nb * BATCH,), f32),
            pltpu.VMEM_SHARED((NP,), f32),
        ],
    )
    def deg_kernel(dst_hbm, ew_hbm, z_hbm, out_hbm, dst_v, ew_v, acc):
        c = lax.axis_index("c")
        s = lax.axis_index("s")
        w = c * NS + s
        pltpu.sync_copy(z_hbm.at[pl.ds(s * NPS, NPS)], acc.at[pl.ds(s * NPS, NPS)])
        pltpu.sync_copy(dst_hbm.at[w], dst_v)
        pltpu.sync_copy(ew_hbm.at[w], ew_v)
        plsc.subcore_barrier()

        def body(g, carry):
            pltpu.sync_copy(ew_v.at[pl.ds(g * BATCH, BATCH)],
                            acc.at[dst_v.at[g]], add=True)
            return carry

        lax.fori_loop(0, nb, body, 0)
        plsc.subcore_barrier()
        pltpu.sync_copy(acc.at[pl.ds(s * NPS, NPS)],
                        out_hbm.at[c, pl.ds(s * NPS, NPS)])

    return deg_kernel


def _make_agg_kernel(NP, nb, H):
    NPS = NP // NS
    NRND = nb // NBUF

    @functools.partial(
        pl.kernel,
        out_type=jax.ShapeDtypeStruct((NC, NP, H), f32),
        mesh=_sc_mesh(),
        compiler_params=pltpu.CompilerParams(needs_layout_passes=False,
                                             use_tc_tiling_on_sc=False),
        scratch_types=[
            pltpu.VMEM((nb, BATCH), i32),
            pltpu.VMEM((nb, BATCH), i32),
            pltpu.VMEM((nb * BATCH,), f32),
            pltpu.VMEM((NBUF, BATCH, H), f32),
            pltpu.VMEM((NBUF, BATCH, H), f32),
            pltpu.VMEM_SHARED((NP, H), f32),
        ] + [pltpu.SemaphoreType.DMA] * (2 * NBUF),
    )
    def agg_kernel(at_hbm, src_hbm, dst_hbm, ew_hbm, z_hbm, out_hbm,
                   src_v, dst_v, ew_v, rows_g, rows_s, acc, *sems):
        gsems = sems[:NBUF]
        ssems = sems[NBUF:]
        c = lax.axis_index("c")
        s = lax.axis_index("s")
        w = c * NS + s
        pltpu.sync_copy(z_hbm.at[pl.ds(s * NPS, NPS)], acc.at[pl.ds(s * NPS, NPS)])
        pltpu.sync_copy(src_hbm.at[w], src_v)
        pltpu.sync_copy(dst_hbm.at[w], dst_v)
        pltpu.sync_copy(ew_hbm.at[w], ew_v)
        plsc.subcore_barrier()

        for b in range(NBUF):
            pltpu.async_copy(at_hbm.at[src_v.at[b]], rows_g.at[b], gsems[b])

        def round_body(go, carry):
            for b in range(NBUF):
                g = go * NBUF + b
                pltpu.make_async_copy(at_hbm.at[src_v.at[g]], rows_g.at[b],
                                      gsems[b]).wait()

                @pl.when(go > 0)
                def _():
                    pltpu.make_async_copy(rows_s.at[b], acc.at[dst_v.at[g]],
                                          ssems[b]).wait()

                base = g * BATCH
                for j in range(BATCH):
                    scale = plsc.load_gather(
                        ew_v, [jnp.full((LANES,), base + j, i32)])
                    rows_s[b, j, :] = rows_g[b, j, :] * scale
                pltpu.async_copy(rows_s.at[b], acc.at[dst_v.at[g]],
                                 ssems[b], add=True)

                @pl.when(go < NRND - 1)
                def _():
                    pltpu.async_copy(at_hbm.at[src_v.at[g + NBUF]],
                                     rows_g.at[b], gsems[b])
            return carry

        lax.fori_loop(0, NRND, round_body, 0)
        for b in range(NBUF):
            g = (NRND - 1) * NBUF + b
            pltpu.make_async_copy(rows_s.at[b], acc.at[dst_v.at[g]],
                                  ssems[b]).wait()
        plsc.subcore_barrier()
        pltpu.sync_copy(acc.at[pl.ds(s * NPS, NPS)],
                        out_hbm.at[c, pl.ds(s * NPS, NPS)])

    return agg_kernel


def _tc_first(x, W1, d0, d1, R):
    N, D = x.shape
    H = W1.shape[1]

    def body(x_ref, w_ref, d0_ref, d1_ref, at_ref, dinv_ref):
        deg = d0_ref[...] + d1_ref[...] + 1.0
        dinv = lax.rsqrt(deg)
        a = jnp.dot(x_ref[...], w_ref[...], preferred_element_type=f32)
        at_ref[...] = a * dinv
        dinv_ref[...] = dinv

    return pl.pallas_call(
        body,
        grid=(N // R,),
        in_specs=[
            pl.BlockSpec((R, D), lambda i: (i, 0)),
            pl.BlockSpec((D, H), lambda i: (0, 0)),
            pl.BlockSpec((R, 1), lambda i: (i, 0)),
            pl.BlockSpec((R, 1), lambda i: (i, 0)),
        ],
        out_specs=[
            pl.BlockSpec((R, H), lambda i: (i, 0)),
            pl.BlockSpec((R, 1), lambda i: (i, 0)),
        ],
        out_shape=[jax.ShapeDtypeStruct((N, H), f32),
                   jax.ShapeDtypeStruct((N, 1), f32)],
    )(x, W1, d0, d1)


def _tc_combine(s0, s1, at, dinv, b, Wn, R):
    N, H = at.shape

    def body(s0_ref, s1_ref, at_ref, dinv_ref, b_ref, w_ref, out_ref):
        o = dinv_ref[...] * (s0_ref[...] + s1_ref[...] + at_ref[...]) + b_ref[...]
        h = jnp.dot(o, w_ref[...], preferred_element_type=f32)
        out_ref[...] = dinv_ref[...] * h

    return pl.pallas_call(
        body,
        grid=(N // R,),
        in_specs=[
            pl.BlockSpec((R, H), lambda i: (i, 0)),
            pl.BlockSpec((R, H), lambda i: (i, 0)),
            pl.BlockSpec((R, H), lambda i: (i, 0)),
            pl.BlockSpec((R, 1), lambda i: (i, 0)),
            pl.BlockSpec((1, H), lambda i: (0, 0)),
            pl.BlockSpec((H, H), lambda i: (0, 0)),
        ],
        out_specs=pl.BlockSpec((R, H), lambda i: (i, 0)),
        out_shape=jax.ShapeDtypeStruct((N, H), f32),
    )(s0, s1, at, dinv, b, Wn)


def _tc_final(s0, s1, at, dinv, b, Wlin, blin, R):
    N, H = at.shape
    grid = N // R

    def body(s0_ref, s1_ref, at_ref, dinv_ref, b_ref, wl_ref, bl_ref,
             out_ref, acc):
        i = pl.program_id(0)

        @pl.when(i == 0)
        def _():
            acc[...] = jnp.zeros_like(acc)

        o = dinv_ref[...] * (s0_ref[...] + s1_ref[...] + at_ref[...]) + b_ref[...]
        acc[...] += jnp.sum(o, axis=0, keepdims=True)

        @pl.when(i == grid - 1)
        def _():
            g = acc[...] * (1.0 / N)
            out_ref[...] = jnp.dot(g, wl_ref[...],
                                   preferred_element_type=f32) + bl_ref[...]

    return pl.pallas_call(
        body,
        grid=(grid,),
        in_specs=[
            pl.BlockSpec((R, H), lambda i: (i, 0)),
            pl.BlockSpec((R, H), lambda i: (i, 0)),
            pl.BlockSpec((R, H), lambda i: (i, 0)),
            pl.BlockSpec((R, 1), lambda i: (i, 0)),
            pl.BlockSpec((1, H), lambda i: (0, 0)),
            pl.BlockSpec((H, 1), lambda i: (0, 0)),
            pl.BlockSpec((1, 1), lambda i: (0, 0)),
        ],
        out_specs=pl.BlockSpec((1, 1), lambda i: (0, 0)),
        out_shape=jax.ShapeDtypeStruct((1, 1), f32),
        scratch_shapes=[pltpu.VMEM((1, H), f32)],
    )(s0, s1, at, dinv, b, Wlin, blin)


def kernel(x, edge_index, edge_attr, W1, b1, W2, b2, W3, b3, Wlin, blin):
    N, D = x.shape
    H = W1.shape[1]
    E = edge_index.shape[1]
    assert H == LANES

    NP = -(-N // (NS * LANES)) * (NS * LANES)
    EP = -(-E // (NW * BATCH * NBUF)) * (NW * BATCH * NBUF)
    nb = EP // (NW * BATCH)
    R = 2000
    assert N % R == 0

    src = edge_index[0]
    dst = edge_index[1]
    ew = edge_attr.astype(f32)
    pad = EP - E
    if pad:
        zi = jnp.zeros((pad,), i32)
        src = jnp.concatenate([src, zi])
        dst = jnp.concatenate([dst, zi])
        ew = jnp.concatenate([ew, jnp.zeros((pad,), f32)])
    src3 = src.reshape(NW, nb, BATCH)
    dst3 = dst.reshape(NW, nb, BATCH)
    ew3 = ew.reshape(NW, nb * BATCH)
    z1 = jnp.zeros((NP,), f32)
    z2 = jnp.zeros((NP, H), f32)

    deg_k = _make_deg_kernel(NP, nb)
    agg_k = _make_agg_kernel(NP, nb, H)

    degp = deg_k(dst3, ew3, z1)
    d0 = degp[0, :N, None]
    d1 = degp[1, :N, None]

    at1, dinv = _tc_first(x, W1, d0, d1, R)
    S = agg_k(at1, src3, dst3, ew3, z2)
    at2 = _tc_combine(S[0, :N], S[1, :N], at1, dinv, b1.reshape(1, H), W2, R)
    S = agg_k(at2, src3, dst3, ew3, z2)
    at3 = _tc_combine(S[0, :N], S[1, :N], at2, dinv, b2.reshape(1, H), W3, R)
    S = agg_k(at3, src3, dst3, ew3, z2)
    return _tc_final(S[0, :N], S[1, :N], at3, dinv, b3.reshape(1, H),
                     Wlin, blin.reshape(1, 1), R)


# speedup vs baseline: 31.6074x; 1.4550x over previous
import functools

import jax
import jax.numpy as jnp
from jax import lax
from jax.experimental import pallas as pl
from jax.experimental.pallas import tpu as pltpu
from jax.experimental.pallas import tpu_sc as plsc

NC = 2
NS = 16
LANES = 16
NW = NC * NS
BATCH = 128
NBUF = 4

f32 = jnp.float32
i32 = jnp.int32


def _lane_bcast(v, t):
    return lax.gather(
        v, jnp.full((LANES, 1), t, i32),
        dimension_numbers=lax.GatherDimensionNumbers(
            offset_dims=(), collapsed_slice_dims=(0,), start_index_map=(0,)),
        slice_sizes=(1,),
        mode=lax.GatherScatterMode.PROMISE_IN_BOUNDS)


def _sc_mesh():
    return plsc.VectorSubcoreMesh(core_axis_name="c", subcore_axis_name="s",
                                  num_cores=NC, num_subcores=NS)


def _make_deg_kernel(NP, nb):
    NPS = NP // NS

    @functools.partial(
        pl.kernel,
        out_type=jax.ShapeDtypeStruct((NC, NP), f32),
        mesh=_sc_mesh(),
        compiler_params=pltpu.CompilerParams(needs_layout_passes=False,
                                             use_tc_tiling_on_sc=False),
        scratch_types=[
            pltpu.VMEM((nb, BATCH), i32),
            pltpu.VMEM((nb * BATCH,), f32),
            pltpu.VMEM_SHARED((NP,), f32),
        ],
    )
    def deg_kernel(dst_hbm, ew_hbm, z_hbm, out_hbm, dst_v, ew_v, acc):
        c = lax.axis_index("c")
        s = lax.axis_index("s")
        w = c * NS + s
        pltpu.sync_copy(z_hbm.at[pl.ds(s * NPS, NPS)], acc.at[pl.ds(s * NPS, NPS)])
        pltpu.sync_copy(dst_hbm.at[w], dst_v)
        pltpu.sync_copy(ew_hbm.at[w], ew_v)
        plsc.subcore_barrier()

        def body(g, carry):
            pltpu.sync_copy(ew_v.at[pl.ds(g * BATCH, BATCH)],
                            acc.at[dst_v.at[g]], add=True)
            return carry

        lax.fori_loop(0, nb, body, 0)
        plsc.subcore_barrier()
        pltpu.sync_copy(acc.at[pl.ds(s * NPS, NPS)],
                        out_hbm.at[c, pl.ds(s * NPS, NPS)])

    return deg_kernel


def _make_agg_kernel(NP, nb, H):
    NPS = NP // NS
    NRND = nb // NBUF

    @functools.partial(
        pl.kernel,
        out_type=jax.ShapeDtypeStruct((NC, NP, H), f32),
        mesh=_sc_mesh(),
        compiler_params=pltpu.CompilerParams(needs_layout_passes=False,
                                             use_tc_tiling_on_sc=False),
        scratch_types=[
            pltpu.VMEM((nb, BATCH), i32),
            pltpu.VMEM((nb, BATCH), i32),
            pltpu.VMEM((nb * BATCH,), f32),
            pltpu.VMEM((NBUF, BATCH, H), f32),
            pltpu.VMEM((NBUF, BATCH, H), f32),
            pltpu.VMEM_SHARED((NP, H), f32),
        ] + [pltpu.SemaphoreType.DMA] * (2 * NBUF),
    )
    def agg_kernel(at_hbm, src_hbm, dst_hbm, ew_hbm, z_hbm, out_hbm,
                   src_v, dst_v, ew_v, rows_g, rows_s, acc, *sems):
        gsems = sems[:NBUF]
        ssems = sems[NBUF:]
        c = lax.axis_index("c")
        s = lax.axis_index("s")
        w = c * NS + s
        pltpu.sync_copy(z_hbm.at[pl.ds(s * NPS, NPS)], acc.at[pl.ds(s * NPS, NPS)])
        pltpu.sync_copy(src_hbm.at[w], src_v)
        pltpu.sync_copy(dst_hbm.at[w], dst_v)
        pltpu.sync_copy(ew_hbm.at[w], ew_v)
        plsc.subcore_barrier()

        for b in range(NBUF):
            pltpu.async_copy(at_hbm.at[src_v.at[b]], rows_g.at[b], gsems[b])

        def round_body(go, carry):
            for b in range(NBUF):
                g = go * NBUF + b
                pltpu.make_async_copy(at_hbm.at[src_v.at[g]], rows_g.at[b],
                                      gsems[b]).wait()

                @pl.when(go > 0)
                def _():
                    pltpu.make_async_copy(rows_s.at[b], acc.at[dst_v.at[g]],
                                          ssems[b]).wait()

                base = g * BATCH
                for jj in range(0, BATCH, LANES):
                    nv = ew_v[pl.ds(base + jj, LANES)]
                    for t in range(LANES):
                        scale = _lane_bcast(nv, t)
                        j = jj + t
                        rows_s[b, j, :] = rows_g[b, j, :] * scale
                pltpu.async_copy(rows_s.at[b], acc.at[dst_v.at[g]],
                                 ssems[b], add=True)

                @pl.when(go < NRND - 1)
                def _():
                    pltpu.async_copy(at_hbm.at[src_v.at[g + NBUF]],
                                     rows_g.at[b], gsems[b])
            return carry

        lax.fori_loop(0, NRND, round_body, 0)
        for b in range(NBUF):
            g = (NRND - 1) * NBUF + b
            pltpu.make_async_copy(rows_s.at[b], acc.at[dst_v.at[g]],
                                  ssems[b]).wait()
        plsc.subcore_barrier()
        pltpu.sync_copy(acc.at[pl.ds(s * NPS, NPS)],
                        out_hbm.at[c, pl.ds(s * NPS, NPS)])

    return agg_kernel


def _tc_first(x, W1, d0, d1, R):
    N, D = x.shape
    H = W1.shape[1]

    def body(x_ref, w_ref, d0_ref, d1_ref, at_ref, dinv_ref):
        deg = d0_ref[...] + d1_ref[...] + 1.0
        dinv = lax.rsqrt(deg)
        a = jnp.dot(x_ref[...], w_ref[...], preferred_element_type=f32)
        at_ref[...] = a * dinv
        dinv_ref[...] = dinv

    return pl.pallas_call(
        body,
        grid=(N // R,),
        in_specs=[
            pl.BlockSpec((R, D), lambda i: (i, 0)),
            pl.BlockSpec((D, H), lambda i: (0, 0)),
            pl.BlockSpec((R, 1), lambda i: (i, 0)),
            pl.BlockSpec((R, 1), lambda i: (i, 0)),
        ],
        out_specs=[
            pl.BlockSpec((R, H), lambda i: (i, 0)),
            pl.BlockSpec((R, 1), lambda i: (i, 0)),
        ],
        out_shape=[jax.ShapeDtypeStruct((N, H), f32),
                   jax.ShapeDtypeStruct((N, 1), f32)],
    )(x, W1, d0, d1)


def _tc_combine(s0, s1, at, dinv, b, Wn, R):
    N, H = at.shape

    def body(s0_ref, s1_ref, at_ref, dinv_ref, b_ref, w_ref, out_ref):
        o = dinv_ref[...] * (s0_ref[...] + s1_ref[...] + at_ref[...]) + b_ref[...]
        h = jnp.dot(o, w_ref[...], preferred_element_type=f32)
        out_ref[...] = dinv_ref[...] * h

    return pl.pallas_call(
        body,
        grid=(N // R,),
        in_specs=[
            pl.BlockSpec((R, H), lambda i: (i, 0)),
            pl.BlockSpec((R, H), lambda i: (i, 0)),
            pl.BlockSpec((R, H), lambda i: (i, 0)),
            pl.BlockSpec((R, 1), lambda i: (i, 0)),
            pl.BlockSpec((1, H), lambda i: (0, 0)),
            pl.BlockSpec((H, H), lambda i: (0, 0)),
        ],
        out_specs=pl.BlockSpec((R, H), lambda i: (i, 0)),
        out_shape=jax.ShapeDtypeStruct((N, H), f32),
    )(s0, s1, at, dinv, b, Wn)


def _tc_final(s0, s1, at, dinv, b, Wlin, blin, R):
    N, H = at.shape
    grid = N // R

    def body(s0_ref, s1_ref, at_ref, dinv_ref, b_ref, wl_ref, bl_ref,
             out_ref, acc):
        i = pl.program_id(0)

        @pl.when(i == 0)
        def _():
            acc[...] = jnp.zeros_like(acc)

        o = dinv_ref[...] * (s0_ref[...] + s1_ref[...] + at_ref[...]) + b_ref[...]
        acc[...] += jnp.sum(o, axis=0, keepdims=True)

        @pl.when(i == grid - 1)
        def _():
            g = acc[...] * (1.0 / N)
            out_ref[...] = jnp.dot(g, wl_ref[...],
                                   preferred_element_type=f32) + bl_ref[...]

    return pl.pallas_call(
        body,
        grid=(grid,),
        in_specs=[
            pl.BlockSpec((R, H), lambda i: (i, 0)),
            pl.BlockSpec((R, H), lambda i: (i, 0)),
            pl.BlockSpec((R, H), lambda i: (i, 0)),
            pl.BlockSpec((R, 1), lambda i: (i, 0)),
            pl.BlockSpec((1, H), lambda i: (0, 0)),
            pl.BlockSpec((H, 1), lambda i: (0, 0)),
            pl.BlockSpec((1, 1), lambda i: (0, 0)),
        ],
        out_specs=pl.BlockSpec((1, 1), lambda i: (0, 0)),
        out_shape=jax.ShapeDtypeStruct((1, 1), f32),
        scratch_shapes=[pltpu.VMEM((1, H), f32)],
    )(s0, s1, at, dinv, b, Wlin, blin)


def kernel(x, edge_index, edge_attr, W1, b1, W2, b2, W3, b3, Wlin, blin):
    N, D = x.shape
    H = W1.shape[1]
    E = edge_index.shape[1]
    assert H == LANES

    NP = -(-N // (NS * LANES)) * (NS * LANES)
    EP = -(-E // (NW * BATCH * NBUF)) * (NW * BATCH * NBUF)
    nb = EP // (NW * BATCH)
    R = 2000
    assert N % R == 0

    src = edge_index[0]
    dst = edge_index[1]
    ew = edge_attr.astype(f32)
    pad = EP - E
    if pad:
        zi = jnp.zeros((pad,), i32)
        src = jnp.concatenate([src, zi])
        dst = jnp.concatenate([dst, zi])
        ew = jnp.concatenate([ew, jnp.zeros((pad,), f32)])
    src3 = src.reshape(NW, nb, BATCH)
    dst3 = dst.reshape(NW, nb, BATCH)
    ew3 = ew.reshape(NW, nb * BATCH)
    z1 = jnp.zeros((NP,), f32)
    z2 = jnp.zeros((NP, H), f32)

    deg_k = _make_deg_kernel(NP, nb)
    agg_k = _make_agg_kernel(NP, nb, H)

    degp = deg_k(dst3, ew3, z1)
    d0 = degp[0, :N, None]
    d1 = degp[1, :N, None]

    at1, dinv = _tc_first(x, W1, d0, d1, R)
    S = agg_k(at1, src3, dst3, ew3, z2)
    at2 = _tc_combine(S[0, :N], S[1, :N], at1, dinv, b1.reshape(1, H), W2, R)
    S = agg_k(at2, src3, dst3, ew3, z2)
    at3 = _tc_combine(S[0, :N], S[1, :N], at2, dinv, b2.reshape(1, H), W3, R)
    S = agg_k(at3, src3, dst3, ew3, z2)
    return _tc_final(S[0, :N], S[1, :N], at3, dinv, b3.reshape(1, H),
                     Wlin, blin.reshape(1, 1), R)


# speedup vs baseline: 47.4384x; 1.5009x over previous
import functools

import jax
import jax.numpy as jnp
from jax import lax
from jax.experimental import pallas as pl
from jax.experimental.pallas import tpu as pltpu
from jax.experimental.pallas import tpu_sc as plsc

NC = 2
NS = 16
LANES = 16
NW = NC * NS
BATCH = 128
NBUF = 4

f32 = jnp.float32
i32 = jnp.int32


def _lane_bcast(v, t):
    return lax.gather(
        v, jnp.full((LANES, 1), t, i32),
        dimension_numbers=lax.GatherDimensionNumbers(
            offset_dims=(), collapsed_slice_dims=(0,), start_index_map=(0,)),
        slice_sizes=(1,),
        mode=lax.GatherScatterMode.PROMISE_IN_BOUNDS)


def _sc_mesh():
    return plsc.VectorSubcoreMesh(core_axis_name="c", subcore_axis_name="s",
                                  num_cores=NC, num_subcores=NS)


def _make_deg_kernel(NP, nb):
    NPS = NP // NS

    @functools.partial(
        pl.kernel,
        out_type=jax.ShapeDtypeStruct((NC, NP), f32),
        mesh=_sc_mesh(),
        compiler_params=pltpu.CompilerParams(needs_layout_passes=False,
                                             use_tc_tiling_on_sc=False),
        scratch_types=[
            pltpu.VMEM((nb, BATCH), i32),
            pltpu.VMEM((nb * BATCH,), f32),
            pltpu.VMEM_SHARED((NP,), f32),
        ],
    )
    def deg_kernel(dst_hbm, ew_hbm, z_hbm, out_hbm, dst_v, ew_v, acc):
        c = lax.axis_index("c")
        s = lax.axis_index("s")
        w = c * NS + s
        pltpu.sync_copy(z_hbm.at[pl.ds(s * NPS, NPS)], acc.at[pl.ds(s * NPS, NPS)])
        pltpu.sync_copy(dst_hbm.at[w], dst_v)
        pltpu.sync_copy(ew_hbm.at[w], ew_v)
        plsc.subcore_barrier()

        def body(g, carry):
            pltpu.sync_copy(ew_v.at[pl.ds(g * BATCH, BATCH)],
                            acc.at[dst_v.at[g]], add=True)
            return carry

        lax.fori_loop(0, nb, body, 0)
        plsc.subcore_barrier()
        pltpu.sync_copy(acc.at[pl.ds(s * NPS, NPS)],
                        out_hbm.at[c, pl.ds(s * NPS, NPS)])

    return deg_kernel


def _make_agg_kernel(NP, nb, H, NROWS_STATIC):
    NPS = NP // NS
    NRND = nb // NBUF

    @functools.partial(
        pl.kernel,
        out_type=jax.ShapeDtypeStruct((NC, NP, H), f32),
        mesh=_sc_mesh(),
        compiler_params=pltpu.CompilerParams(needs_layout_passes=False,
                                             use_tc_tiling_on_sc=False),
        scratch_types=[
            pltpu.VMEM((nb, BATCH), i32),
            pltpu.VMEM((nb, BATCH), i32),
            pltpu.VMEM((nb * BATCH,), f32),
            pltpu.VMEM((NBUF, BATCH, H), f32),
            pltpu.VMEM((NBUF, BATCH, H), f32),
            pltpu.VMEM_SHARED((NP, H), f32),
            pltpu.VMEM_SHARED((NROWS_STATIC, H), f32),
        ] + [pltpu.SemaphoreType.DMA] * (2 * NBUF),
    )
    def agg_kernel(at_hbm, src_hbm, dst_hbm, ew_hbm, z_hbm, out_hbm,
                   src_v, dst_v, ew_v, rows_g, rows_s, acc, at_sh, *sems):
        gsems = sems[:NBUF]
        ssems = sems[NBUF:]
        c = lax.axis_index("c")
        s = lax.axis_index("s")
        w = c * NS + s
        NROWS = at_hbm.shape[0]
        RPT = NROWS // NS
        pltpu.sync_copy(z_hbm.at[pl.ds(s * NPS, NPS)], acc.at[pl.ds(s * NPS, NPS)])
        pltpu.sync_copy(at_hbm.at[pl.ds(s * RPT, RPT)],
                        at_sh.at[pl.ds(s * RPT, RPT)])
        pltpu.sync_copy(src_hbm.at[w], src_v)
        pltpu.sync_copy(dst_hbm.at[w], dst_v)
        pltpu.sync_copy(ew_hbm.at[w], ew_v)
        plsc.subcore_barrier()

        for b in range(NBUF):
            pltpu.async_copy(at_sh.at[src_v.at[b]], rows_g.at[b], gsems[b])

        def round_body(go, carry):
            for b in range(NBUF):
                g = go * NBUF + b
                pltpu.make_async_copy(at_sh.at[src_v.at[g]], rows_g.at[b],
                                      gsems[b]).wait()

                @pl.when(go > 0)
                def _():
                    pltpu.make_async_copy(rows_s.at[b], acc.at[dst_v.at[g]],
                                          ssems[b]).wait()

                base = g * BATCH
                for jj in range(0, BATCH, LANES):
                    nv = ew_v[pl.ds(base + jj, LANES)]
                    for t in range(LANES):
                        scale = _lane_bcast(nv, t)
                        j = jj + t
                        rows_s[b, j, :] = rows_g[b, j, :] * scale
                pltpu.async_copy(rows_s.at[b], acc.at[dst_v.at[g]],
                                 ssems[b], add=True)

                @pl.when(go < NRND - 1)
                def _():
                    pltpu.async_copy(at_sh.at[src_v.at[g + NBUF]],
                                     rows_g.at[b], gsems[b])
            return carry

        lax.fori_loop(0, NRND, round_body, 0)
        for b in range(NBUF):
            g = (NRND - 1) * NBUF + b
            pltpu.make_async_copy(rows_s.at[b], acc.at[dst_v.at[g]],
                                  ssems[b]).wait()
        plsc.subcore_barrier()
        pltpu.sync_copy(acc.at[pl.ds(s * NPS, NPS)],
                        out_hbm.at[c, pl.ds(s * NPS, NPS)])

    return agg_kernel


def _tc_first(x, W1, d0, d1, R):
    N, D = x.shape
    H = W1.shape[1]

    def body(x_ref, w_ref, d0_ref, d1_ref, at_ref, dinv_ref):
        deg = d0_ref[...] + d1_ref[...] + 1.0
        dinv = lax.rsqrt(deg)
        a = jnp.dot(x_ref[...], w_ref[...], preferred_element_type=f32)
        at_ref[...] = a * dinv
        dinv_ref[...] = dinv

    return pl.pallas_call(
        body,
        grid=(N // R,),
        in_specs=[
            pl.BlockSpec((R, D), lambda i: (i, 0)),
            pl.BlockSpec((D, H), lambda i: (0, 0)),
            pl.BlockSpec((R, 1), lambda i: (i, 0)),
            pl.BlockSpec((R, 1), lambda i: (i, 0)),
        ],
        out_specs=[
            pl.BlockSpec((R, H), lambda i: (i, 0)),
            pl.BlockSpec((R, 1), lambda i: (i, 0)),
        ],
        out_shape=[jax.ShapeDtypeStruct((N, H), f32),
                   jax.ShapeDtypeStruct((N, 1), f32)],
    )(x, W1, d0, d1)


def _tc_combine(s0, s1, at, dinv, b, Wn, R):
    N, H = at.shape

    def body(s0_ref, s1_ref, at_ref, dinv_ref, b_ref, w_ref, out_ref):
        o = dinv_ref[...] * (s0_ref[...] + s1_ref[...] + at_ref[...]) + b_ref[...]
        h = jnp.dot(o, w_ref[...], preferred_element_type=f32)
        out_ref[...] = dinv_ref[...] * h

    return pl.pallas_call(
        body,
        grid=(N // R,),
        in_specs=[
            pl.BlockSpec((R, H), lambda i: (i, 0)),
            pl.BlockSpec((R, H), lambda i: (i, 0)),
            pl.BlockSpec((R, H), lambda i: (i, 0)),
            pl.BlockSpec((R, 1), lambda i: (i, 0)),
            pl.BlockSpec((1, H), lambda i: (0, 0)),
            pl.BlockSpec((H, H), lambda i: (0, 0)),
        ],
        out_specs=pl.BlockSpec((R, H), lambda i: (i, 0)),
        out_shape=jax.ShapeDtypeStruct((N, H), f32),
    )(s0, s1, at, dinv, b, Wn)


def _tc_final(s0, s1, at, dinv, b, Wlin, blin, R):
    N, H = at.shape
    grid = N // R

    def body(s0_ref, s1_ref, at_ref, dinv_ref, b_ref, wl_ref, bl_ref,
             out_ref, acc):
        i = pl.program_id(0)

        @pl.when(i == 0)
        def _():
            acc[...] = jnp.zeros_like(acc)

        o = dinv_ref[...] * (s0_ref[...] + s1_ref[...] + at_ref[...]) + b_ref[...]
        acc[...] += jnp.sum(o, axis=0, keepdims=True)

        @pl.when(i == grid - 1)
        def _():
            g = acc[...] * (1.0 / N)
            out_ref[...] = jnp.dot(g, wl_ref[...],
                                   preferred_element_type=f32) + bl_ref[...]

    return pl.pallas_call(
        body,
        grid=(grid,),
        in_specs=[
            pl.BlockSpec((R, H), lambda i: (i, 0)),
            pl.BlockSpec((R, H), lambda i: (i, 0)),
            pl.BlockSpec((R, H), lambda i: (i, 0)),
            pl.BlockSpec((R, 1), lambda i: (i, 0)),
            pl.BlockSpec((1, H), lambda i: (0, 0)),
            pl.BlockSpec((H, 1), lambda i: (0, 0)),
            pl.BlockSpec((1, 1), lambda i: (0, 0)),
        ],
        out_specs=pl.BlockSpec((1, 1), lambda i: (0, 0)),
        out_shape=jax.ShapeDtypeStruct((1, 1), f32),
        scratch_shapes=[pltpu.VMEM((1, H), f32)],
    )(s0, s1, at, dinv, b, Wlin, blin)


def kernel(x, edge_index, edge_attr, W1, b1, W2, b2, W3, b3, Wlin, blin):
    N, D = x.shape
    H = W1.shape[1]
    E = edge_index.shape[1]
    assert H == LANES

    NP = -(-N // (NS * LANES)) * (NS * LANES)
    EP = -(-E // (NW * BATCH * NBUF)) * (NW * BATCH * NBUF)
    nb = EP // (NW * BATCH)
    R = 2000
    assert N % R == 0

    src = edge_index[0]
    dst = edge_index[1]
    ew = edge_attr.astype(f32)
    pad = EP - E
    if pad:
        zi = jnp.zeros((pad,), i32)
        src = jnp.concatenate([src, zi])
        dst = jnp.concatenate([dst, zi])
        ew = jnp.concatenate([ew, jnp.zeros((pad,), f32)])
    src3 = src.reshape(NW, nb, BATCH)
    dst3 = dst.reshape(NW, nb, BATCH)
    ew3 = ew.reshape(NW, nb * BATCH)
    z1 = jnp.zeros((NP,), f32)
    z2 = jnp.zeros((NP, H), f32)

    deg_k = _make_deg_kernel(NP, nb)
    agg_k = _make_agg_kernel(NP, nb, H, N)

    degp = deg_k(dst3, ew3, z1)
    d0 = degp[0, :N, None]
    d1 = degp[1, :N, None]

    at1, dinv = _tc_first(x, W1, d0, d1, R)
    S = agg_k(at1, src3, dst3, ew3, z2)
    at2 = _tc_combine(S[0, :N], S[1, :N], at1, dinv, b1.reshape(1, H), W2, R)
    S = agg_k(at2, src3, dst3, ew3, z2)
    at3 = _tc_combine(S[0, :N], S[1, :N], at2, dinv, b2.reshape(1, H), W3, R)
    S = agg_k(at3, src3, dst3, ew3, z2)
    return _tc_final(S[0, :N], S[1, :N], at3, dinv, b3.reshape(1, H),
                     Wlin, blin.reshape(1, 1), R)
